# Initial kernel scaffold; baseline (speedup 1.0000x reference)
import jax
import jax.numpy as jnp
from jax.experimental import pallas as pl


def kernel(pred, gt):
    raise NotImplementedError("write your pallas kernel here")


# speedup vs baseline: 1.8025x; 1.8025x over previous
import numpy as np
import jax
import jax.numpy as jnp
from jax import lax
from jax.experimental import pallas as pl
from jax.experimental.pallas import tpu as pltpu

_TILE = 400
_MP = 2048
_CLS_OFF = 64.0
_T2 = np.float32((2.5 * np.log(3.0)) ** 2)
_BIG_I = np.int32(2 ** 30)


def _loss_body(cx_ref, cy_ref, cz_ref, conf_ref, tx_ref, ty_ref, tz_ref,
               out_ref, colmin, colarg, rowmin, acc):
    b = pl.program_id(0)
    p = pl.program_id(1)
    t = pl.program_id(2)
    nb = pl.num_programs(0)
    nt = pl.num_programs(2)

    @pl.when((b == 0) & (p == 0) & (t == 0))
    def _init_acc():
        acc[...] = jnp.zeros_like(acc)

    @pl.when(p == 0)
    def _phase0():
        @pl.when(t == 0)
        def _reset():
            colmin[...] = jnp.full_like(colmin, jnp.inf)
            colarg[...] = jnp.zeros_like(colarg)

        cxv = cx_ref[0, 0]
        cyv = cy_ref[0, 0]
        czv = cz_ref[0, 0]
        txv = tx_ref[0]
        tyv = ty_ref[0]
        tzv = tz_ref[0]

        dx = cxv - txv
        dy = cyv - tyv
        dz = czv - tzv
        d2 = dx * dx + dy * dy
        rowmin[t] = jnp.min(d2, axis=1, keepdims=True)

        d2c = d2 + dz * dz
        tmin = jnp.min(d2c, axis=0, keepdims=True)
        gidx = (lax.broadcasted_iota(jnp.int32, (_TILE, _MP), 0)
                + t * _TILE)
        targ = jnp.min(jnp.where(d2c == tmin, gidx, _BIG_I),
                       axis=0, keepdims=True)
        better = tmin < colmin[...]
        colarg[...] = jnp.where(better, targ, colarg[...])
        colmin[...] = jnp.where(better, tmin, colmin[...])

    @pl.when(p == 1)
    def _phase1():
        validm = colmin[...] <= _T2
        argv = colarg[...]
        gidx = (lax.broadcasted_iota(jnp.int32, (_TILE, _MP), 0)
                + t * _TILE)
        hit = jnp.any((gidx == argv) & validm, axis=1, keepdims=True)
        confv = conf_ref[0, 0]
        obj_c = (jnp.sum(jax.nn.softplus(confv))
                 - jnp.sum(jnp.where(hit, confv, 0.0)))
        d = jnp.sqrt(rowmin[t] + 1e-12)
        sc = 2.0 * jax.nn.sigmoid(-(d / 2.5))
        reg_c = jnp.sum(jnp.where(hit, sc, 0.0))

        lane = lax.broadcasted_iota(jnp.int32, (1, 128), 1)
        acc[...] = (acc[...]
                    + jnp.where(lane == 0, obj_c, 0.0)
                    + jnp.where(lane == 1, reg_c, 0.0))

        @pl.when((b == nb - 1) & (t == nt - 1))
        def _emit():
            denom = jnp.asarray(nb * nt * _TILE, jnp.float32)
            tot = acc[...]
            out_ref[...] = jnp.where(lane == 0, tot / denom,
                                     1.0 - tot / denom)


def kernel(pred, gt):
    B, N, _ = pred.shape
    M = gt.shape[1]
    T = N // _TILE

    cls = pred[:, :, 0]
    cx = pred[:, :, 1].reshape(B, T, _TILE, 1)
    cy = pred[:, :, 2].reshape(B, T, _TILE, 1)
    cz = (cls * _CLS_OFF).reshape(B, T, _TILE, 1)
    conf = pred[:, :, 3].reshape(B, T, _TILE, 1)

    pad = _MP - M
    far = jnp.full((B, pad), 1e6, dtype=jnp.float32)
    tx = jnp.concatenate([gt[:, :, 1], far], axis=1).reshape(B, 1, _MP)
    ty = jnp.concatenate([gt[:, :, 2], far], axis=1).reshape(B, 1, _MP)
    tz = jnp.concatenate([gt[:, :, 0] * _CLS_OFF, far], axis=1).reshape(B, 1, _MP)

    pred_spec = pl.BlockSpec((1, 1, _TILE, 1), lambda b, p, t: (b, t, 0, 0))
    tgt_spec = pl.BlockSpec((1, 1, _MP), lambda b, p, t: (b, 0, 0))

    out = pl.pallas_call(
        _loss_body,
        grid=(B, 2, T),
        in_specs=[pred_spec, pred_spec, pred_spec, pred_spec,
                  tgt_spec, tgt_spec, tgt_spec],
        out_specs=pl.BlockSpec((1, 128), lambda b, p, t: (0, 0)),
        out_shape=jax.ShapeDtypeStruct((1, 128), jnp.float32),
        scratch_shapes=[
            pltpu.VMEM((1, _MP), jnp.float32),
            pltpu.VMEM((1, _MP), jnp.int32),
            pltpu.VMEM((T, _TILE, 1), jnp.float32),
            pltpu.VMEM((1, 128), jnp.float32),
        ],
    )(cx, cy, cz, conf, tx, ty, tz)
    return out[0, 0], out[0, 1]


# speedup vs baseline: 1.9684x; 1.0921x over previous
import numpy as np
import jax
import jax.numpy as jnp
from jax import lax
from jax.experimental import pallas as pl
from jax.experimental.pallas import tpu as pltpu

_TILE = 800
_MP = 2048
_CLS_OFF = 64.0
_T2 = np.float32((2.5 * np.log(3.0)) ** 2)
_BIG_I = np.int32(2 ** 30)


def _loss_body(cx_ref, cy_ref, cz_ref, conf_ref, tx_ref, ty_ref, tz_ref,
               out_ref, colmin, colarg, rowmin, acc):
    b = pl.program_id(0)
    p = pl.program_id(1)
    t = pl.program_id(2)
    nb = pl.num_programs(0)
    nt = pl.num_programs(2)

    @pl.when((b == 0) & (p == 0) & (t == 0))
    def _init_acc():
        acc[...] = jnp.zeros_like(acc)

    @pl.when(p == 0)
    def _phase0():
        @pl.when(t == 0)
        def _reset():
            colmin[...] = jnp.full_like(colmin, jnp.inf)
            colarg[...] = jnp.zeros_like(colarg)

        cxv = cx_ref[0, 0]
        cyv = cy_ref[0, 0]
        czv = cz_ref[0, 0]
        txv = tx_ref[0]
        tyv = ty_ref[0]
        tzv = tz_ref[0]

        dx = cxv - txv
        dy = cyv - tyv
        dz = czv - tzv
        d2 = dx * dx + dy * dy
        rowmin[t] = jnp.min(d2, axis=1, keepdims=True)

        d2c = d2 + dz * dz
        tmin = jnp.min(d2c, axis=0, keepdims=True)
        lidx = lax.broadcasted_iota(jnp.int32, (_TILE, _MP), 0)
        targ = (jnp.min(jnp.where(d2c == tmin, lidx, _BIG_I),
                        axis=0, keepdims=True)
                + t * _TILE)
        better = tmin < colmin[...]
        colarg[...] = jnp.where(better, targ, colarg[...])
        colmin[...] = jnp.where(better, tmin, colmin[...])

    @pl.when(p == 1)
    def _phase1():
        validm = colmin[...] <= _T2
        argv = jnp.where(validm, colarg[...] - t * _TILE, jnp.int32(-1))
        lidx = lax.broadcasted_iota(jnp.int32, (_TILE, _MP), 0)
        hit = jnp.any(lidx == argv, axis=1, keepdims=True)
        confv = conf_ref[0, 0]
        obj_c = (jnp.sum(jax.nn.softplus(confv))
                 - jnp.sum(jnp.where(hit, confv, 0.0)))
        d = jnp.sqrt(rowmin[t] + 1e-12)
        sc = 2.0 * jax.nn.sigmoid(-(d / 2.5))
        reg_c = jnp.sum(jnp.where(hit, sc, 0.0))

        lane = lax.broadcasted_iota(jnp.int32, (1, 128), 1)
        acc[...] = (acc[...]
                    + jnp.where(lane == 0, obj_c, 0.0)
                    + jnp.where(lane == 1, reg_c, 0.0))

        @pl.when((b == nb - 1) & (t == nt - 1))
        def _emit():
            denom = jnp.asarray(nb * nt * _TILE, jnp.float32)
            tot = acc[...]
            out_ref[...] = jnp.where(lane == 0, tot / denom,
                                     1.0 - tot / denom)


def kernel(pred, gt):
    B, N, _ = pred.shape
    M = gt.shape[1]
    T = N // _TILE

    cls = pred[:, :, 0]
    cx = pred[:, :, 1].reshape(B, T, _TILE, 1)
    cy = pred[:, :, 2].reshape(B, T, _TILE, 1)
    cz = (cls * _CLS_OFF).reshape(B, T, _TILE, 1)
    conf = pred[:, :, 3].reshape(B, T, _TILE, 1)

    pad = _MP - M
    far = jnp.full((B, pad), 1e6, dtype=jnp.float32)
    tx = jnp.concatenate([gt[:, :, 1], far], axis=1).reshape(B, 1, _MP)
    ty = jnp.concatenate([gt[:, :, 2], far], axis=1).reshape(B, 1, _MP)
    tz = jnp.concatenate([gt[:, :, 0] * _CLS_OFF, far], axis=1).reshape(B, 1, _MP)

    pred_spec = pl.BlockSpec((1, 1, _TILE, 1), lambda b, p, t: (b, t, 0, 0))
    tgt_spec = pl.BlockSpec((1, 1, _MP), lambda b, p, t: (b, 0, 0))

    out = pl.pallas_call(
        _loss_body,
        grid=(B, 2, T),
        in_specs=[pred_spec, pred_spec, pred_spec, pred_spec,
                  tgt_spec, tgt_spec, tgt_spec],
        out_specs=pl.BlockSpec((1, 128), lambda b, p, t: (0, 0)),
        out_shape=jax.ShapeDtypeStruct((1, 128), jnp.float32),
        scratch_shapes=[
            pltpu.VMEM((1, _MP), jnp.float32),
            pltpu.VMEM((1, _MP), jnp.int32),
            pltpu.VMEM((T, _TILE, 1), jnp.float32),
            pltpu.VMEM((1, 128), jnp.float32),
        ],
    )(cx, cy, cz, conf, tx, ty, tz)
    return out[0, 0], out[0, 1]


# speedup vs baseline: 2.7783x; 1.4114x over previous
import functools

import numpy as np
import jax
import jax.numpy as jnp
from jax import lax
from jax.experimental import pallas as pl
from jax.experimental.pallas import tpu as pltpu
from jax.experimental.pallas import tpu_sc as plsc

_TILE = 800
_MP = 2048
_NP = 20480
_CLS_OFF = 64.0
_T2 = np.float32((2.5 * np.log(3.0)) ** 2)
_T2_BITS = int(np.float32(_T2).view(np.int32))
_ROW_MASK = 1023
_SC_ROWS = _NP // 32


def _sweep_body(cx_ref, cy_ref, cz_ref, tx_ref, ty_ref, tz_ref,
                rowmin_ref, key_ref, wtile_ref, colkey, wtile):
    b = pl.program_id(0)
    t = pl.program_id(1)
    nt = pl.num_programs(1)

    @pl.when(t == 0)
    def _reset():
        colkey[...] = jnp.full_like(colkey, jnp.inf)
        wtile[...] = jnp.zeros_like(wtile)

    cxv = cx_ref[0, 0]
    cyv = cy_ref[0, 0]
    czv = cz_ref[0, 0]
    txv = tx_ref[0]
    tyv = ty_ref[0]
    tzv = tz_ref[0]

    dx = cxv - txv
    dy = cyv - tyv
    dz = czv - tzv
    d2 = dx * dx + dy * dy
    rowmin_ref[0, 0] = jnp.min(d2, axis=1, keepdims=True)

    d2c = d2 + dz * dz
    lidx = lax.broadcasted_iota(jnp.int32, (_TILE, _MP), 0)
    key = lax.bitcast_convert_type(
        (lax.bitcast_convert_type(d2c, jnp.int32) & jnp.int32(~_ROW_MASK))
        | lidx, jnp.float32)
    kmin = jnp.min(key, axis=0, keepdims=True)
    better = kmin < colkey[...]
    wtile[...] = jnp.where(better, t, wtile[...])
    colkey[...] = jnp.where(better, kmin, colkey[...])

    @pl.when(t == nt - 1)
    def _emit():
        key_ref[0] = lax.bitcast_convert_type(colkey[...], jnp.int32)
        wtile_ref[0] = wtile[...]


def _reduce_body(conf_ref, hits_ref, rowmin_ref, out_ref, acc):
    b = pl.program_id(0)
    nb = pl.num_programs(0)

    @pl.when(b == 0)
    def _init():
        acc[...] = jnp.zeros_like(acc)

    confv = conf_ref[0]
    hitv = hits_ref[0]
    rmv = rowmin_ref[0]
    d = jnp.sqrt(rmv + 1e-12)
    sc = 2.0 * jax.nn.sigmoid(-(d / 2.5))
    obj_c = (jnp.sum(jax.nn.softplus(confv))
             - jnp.sum(jnp.where(hitv > 0.0, confv, 0.0)))
    reg_c = jnp.sum(jnp.where(hitv > 0.0, sc, 0.0))

    lane = lax.broadcasted_iota(jnp.int32, (1, 128), 1)
    acc[...] = (acc[...]
                + jnp.where(lane == 0, obj_c, 0.0)
                + jnp.where(lane == 1, reg_c, 0.0))

    @pl.when(b == nb - 1)
    def _emit():
        denom = jnp.asarray(nb * 20000, jnp.float32)
        tot = acc[...]
        out_ref[...] = jnp.where(lane == 0, tot / denom, 1.0 - tot / denom)


def _sc_scatter_body(key_hbm, wtile_hbm, hits_hbm, keybuf, tilebuf, loc):
    wid = lax.axis_index("s") * 2 + lax.axis_index("c")
    lo = wid * _SC_ROWS
    for b in range(4):
        def _zero(i, carry):
            loc[pl.ds(i * 16, 16)] = jnp.zeros((16,), jnp.float32)
            return carry
        lax.fori_loop(0, _SC_ROWS // 16, _zero, 0)
        pltpu.sync_copy(key_hbm.at[b], keybuf)
        pltpu.sync_copy(wtile_hbm.at[b], tilebuf)

        def _scatter(j, carry):
            k = keybuf[pl.ds(j * 16, 16)]
            w = tilebuf[pl.ds(j * 16, 16)]
            n = w * _TILE + (k & _ROW_MASK)
            valid = (k & jnp.int32(~_ROW_MASK)) <= jnp.int32(_T2_BITS)
            inr = valid & (n >= lo) & (n < lo + _SC_ROWS)
            li = jnp.where(inr, n - lo, 0)
            plsc.store_scatter(loc, [li], jnp.ones((16,), jnp.float32),
                               mask=inr)
            return carry
        lax.fori_loop(0, _MP // 16, _scatter, 0)
        pltpu.sync_copy(loc, hits_hbm.at[b, pl.ds(lo, _SC_ROWS)])


def _make_sc_scatter():
    return pl.kernel(
        _sc_scatter_body,
        mesh=plsc.VectorSubcoreMesh(core_axis_name="c", subcore_axis_name="s"),
        out_type=jax.ShapeDtypeStruct((4, _NP), jnp.float32),
        scratch_types=[
            pltpu.VMEM((_MP,), jnp.int32),
            pltpu.VMEM((_MP,), jnp.int32),
            pltpu.VMEM((_SC_ROWS,), jnp.float32),
        ],
        compiler_params=pltpu.CompilerParams(needs_layout_passes=False),
    )


def kernel(pred, gt):
    B, N, _ = pred.shape
    M = gt.shape[1]
    T = N // _TILE

    cls = pred[:, :, 0]
    cx = pred[:, :, 1].reshape(B, T, _TILE, 1)
    cy = pred[:, :, 2].reshape(B, T, _TILE, 1)
    cz = (cls * _CLS_OFF).reshape(B, T, _TILE, 1)
    conf = pred[:, :, 3]

    pad = _MP - M
    far = jnp.full((B, pad), 1e6, dtype=jnp.float32)
    tx = jnp.concatenate([gt[:, :, 1], far], axis=1).reshape(B, 1, _MP)
    ty = jnp.concatenate([gt[:, :, 2], far], axis=1).reshape(B, 1, _MP)
    tz = jnp.concatenate([gt[:, :, 0] * _CLS_OFF, far], axis=1).reshape(B, 1, _MP)

    pred_spec = pl.BlockSpec((1, 1, _TILE, 1), lambda b, t: (b, t, 0, 0))
    tgt_spec = pl.BlockSpec((1, 1, _MP), lambda b, t: (b, 0, 0))

    rowmin, colkey, wtile = pl.pallas_call(
        _sweep_body,
        grid=(B, T),
        in_specs=[pred_spec, pred_spec, pred_spec,
                  tgt_spec, tgt_spec, tgt_spec],
        out_specs=[pl.BlockSpec((1, 1, _TILE, 1), lambda b, t: (b, t, 0, 0)),
                   pl.BlockSpec((1, 1, _MP), lambda b, t: (b, 0, 0)),
                   pl.BlockSpec((1, 1, _MP), lambda b, t: (b, 0, 0))],
        out_shape=[jax.ShapeDtypeStruct((B, T, _TILE, 1), jnp.float32),
                   jax.ShapeDtypeStruct((B, 1, _MP), jnp.int32),
                   jax.ShapeDtypeStruct((B, 1, _MP), jnp.int32)],
        scratch_shapes=[
            pltpu.VMEM((1, _MP), jnp.float32),
            pltpu.VMEM((1, _MP), jnp.int32),
        ],
    )(cx, cy, cz, tx, ty, tz)

    hits = _make_sc_scatter()(colkey.reshape(B, _MP), wtile.reshape(B, _MP))

    padn = _NP - N
    conf_p = jnp.concatenate(
        [conf, jnp.full((B, padn), -1e30, jnp.float32)], axis=1
    ).reshape(B, 160, 128)
    rowmin_p = jnp.concatenate(
        [rowmin.reshape(B, N), jnp.full((B, padn), 1e30, jnp.float32)], axis=1
    ).reshape(B, 160, 128)
    hits_p = hits.reshape(B, 160, 128)

    red_spec = pl.BlockSpec((1, 160, 128), lambda b: (b, 0, 0))
    out = pl.pallas_call(
        _reduce_body,
        grid=(B,),
        in_specs=[red_spec, red_spec, red_spec],
        out_specs=pl.BlockSpec((1, 128), lambda b: (0, 0)),
        out_shape=jax.ShapeDtypeStruct((1, 128), jnp.float32),
        scratch_shapes=[pltpu.VMEM((1, 128), jnp.float32)],
    )(conf_p, hits_p, rowmin_p)
    return out[0, 0], out[0, 1]


# speedup vs baseline: 3.9937x; 1.4375x over previous
import numpy as np
import jax
import jax.numpy as jnp
from jax import lax
from jax.experimental import pallas as pl
from jax.experimental.pallas import tpu as pltpu
from jax.experimental.pallas import tpu_sc as plsc

_TILE = 1000
_MP = 2048
_NP = 20480
_CLS_OFF = 64.0
_T2 = np.float32((2.5 * np.log(3.0)) ** 2)
_T2_BITS = int(np.float32(_T2).view(np.int32))
_ROW_MASK = 1023
_SC_ROWS = _NP // 32


def _sweep_body(p_ref, g_ref, rowmin_ref, key_ref, wtile_ref, colkey, wtile):
    t = pl.program_id(1)
    nt = pl.num_programs(1)

    @pl.when(t == 0)
    def _reset():
        colkey[...] = jnp.full_like(colkey, jnp.inf)
        wtile[...] = jnp.zeros_like(wtile)

    pv = p_ref[0, 0]
    gv = g_ref[0]
    czv = pv[:, 0:1] * _CLS_OFF
    cxv = pv[:, 1:2]
    cyv = pv[:, 2:3]
    tzv = gv[0:1]
    txv = gv[1:2]
    tyv = gv[2:3]

    dx = cxv - txv
    dy = cyv - tyv
    dz = czv - tzv
    d2 = dx * dx + dy * dy
    rowmin_ref[0, 0] = jnp.min(d2, axis=1, keepdims=True)

    d2c = d2 + dz * dz
    lidx = lax.broadcasted_iota(jnp.int32, (_TILE, _MP), 0)
    key = lax.bitcast_convert_type(
        (lax.bitcast_convert_type(d2c, jnp.int32) & jnp.int32(~_ROW_MASK))
        | lidx, jnp.float32)
    kmin = jnp.min(key, axis=0, keepdims=True)
    better = kmin < colkey[...]
    wtile[...] = jnp.where(better, t, wtile[...])
    colkey[...] = jnp.where(better, kmin, colkey[...])

    @pl.when(t == nt - 1)
    def _emit():
        key_ref[0] = lax.bitcast_convert_type(colkey[...], jnp.int32)
        wtile_ref[0] = wtile[...]


def _reduce_body(conf_ref, hits_ref, rowmin_ref, out_ref, acc):
    b = pl.program_id(0)
    nb = pl.num_programs(0)

    @pl.when(b == 0)
    def _init():
        acc[...] = jnp.zeros_like(acc)

    confv = conf_ref[0]
    hitv = hits_ref[0]
    rmv = rowmin_ref[0]
    d = jnp.sqrt(rmv + 1e-12)
    sc = 2.0 * jax.nn.sigmoid(-(d / 2.5))
    obj_c = (jnp.sum(jax.nn.softplus(confv))
             - jnp.sum(jnp.where(hitv > 0.0, confv, 0.0)))
    reg_c = jnp.sum(jnp.where(hitv > 0.0, sc, 0.0))

    lane = lax.broadcasted_iota(jnp.int32, (1, 128), 1)
    acc[...] = (acc[...]
                + jnp.where(lane == 0, obj_c, 0.0)
                + jnp.where(lane == 1, reg_c, 0.0))

    @pl.when(b == nb - 1)
    def _emit():
        denom = jnp.asarray(nb * 20000, jnp.float32)
        tot = acc[...]
        out_ref[...] = jnp.where(lane == 0, tot / denom, 1.0 - tot / denom)


def _sc_scatter_body(key_hbm, wtile_hbm, hits_hbm, keybuf, tilebuf, loc):
    wid = lax.axis_index("s") * 2 + lax.axis_index("c")
    lo = wid * _SC_ROWS
    for b in range(4):
        def _zero(i, carry):
            loc[pl.ds(i * 16, 16)] = jnp.zeros((16,), jnp.float32)
            return carry
        lax.fori_loop(0, _SC_ROWS // 16, _zero, 0)
        pltpu.sync_copy(key_hbm.at[b], keybuf)
        pltpu.sync_copy(wtile_hbm.at[b], tilebuf)

        def _scatter(j, carry):
            k = keybuf[pl.ds(j * 16, 16)]
            w = tilebuf[pl.ds(j * 16, 16)]
            n = w * _TILE + (k & _ROW_MASK)
            valid = (k & jnp.int32(~_ROW_MASK)) <= jnp.int32(_T2_BITS)
            inr = valid & (n >= lo) & (n < lo + _SC_ROWS)
            li = jnp.where(inr, n - lo, 0)
            plsc.store_scatter(loc, [li], jnp.ones((16,), jnp.float32),
                               mask=inr)
            return carry
        lax.fori_loop(0, _MP // 16, _scatter, 0)
        pltpu.sync_copy(loc, hits_hbm.at[b, pl.ds(lo, _SC_ROWS)])


def _make_sc_scatter():
    return pl.kernel(
        _sc_scatter_body,
        mesh=plsc.VectorSubcoreMesh(core_axis_name="c", subcore_axis_name="s"),
        out_type=jax.ShapeDtypeStruct((4, _NP), jnp.float32),
        scratch_types=[
            pltpu.VMEM((_MP,), jnp.int32),
            pltpu.VMEM((_MP,), jnp.int32),
            pltpu.VMEM((_SC_ROWS,), jnp.float32),
        ],
        compiler_params=pltpu.CompilerParams(needs_layout_passes=False),
    )


def kernel(pred, gt):
    B, N, _ = pred.shape
    M = gt.shape[1]
    T = N // _TILE

    pv = pred.reshape(B, T, _TILE, 4)
    pad = _MP - M
    gscaled = jnp.stack(
        [gt[:, :, 0] * _CLS_OFF, gt[:, :, 1], gt[:, :, 2]], axis=1)
    gv = jnp.concatenate(
        [gscaled, jnp.full((B, 3, pad), 1e6, jnp.float32)], axis=2)

    rowmin, colkey, wtile = pl.pallas_call(
        _sweep_body,
        grid=(B, T),
        in_specs=[pl.BlockSpec((1, 1, _TILE, 4), lambda b, t: (b, t, 0, 0)),
                  pl.BlockSpec((1, 3, _MP), lambda b, t: (b, 0, 0))],
        out_specs=[pl.BlockSpec((1, 1, _TILE, 1), lambda b, t: (b, t, 0, 0)),
                   pl.BlockSpec((1, 1, _MP), lambda b, t: (b, 0, 0)),
                   pl.BlockSpec((1, 1, _MP), lambda b, t: (b, 0, 0))],
        out_shape=[jax.ShapeDtypeStruct((B, T, _TILE, 1), jnp.float32),
                   jax.ShapeDtypeStruct((B, 1, _MP), jnp.int32),
                   jax.ShapeDtypeStruct((B, 1, _MP), jnp.int32)],
        scratch_shapes=[
            pltpu.VMEM((1, _MP), jnp.float32),
            pltpu.VMEM((1, _MP), jnp.int32),
        ],
    )(pv, gv)

    hits = _make_sc_scatter()(colkey.reshape(B, _MP), wtile.reshape(B, _MP))

    padn = _NP - N
    conf_p = jnp.concatenate(
        [pred[:, :, 3], jnp.full((B, padn), -1e30, jnp.float32)], axis=1
    ).reshape(B, 160, 128)
    rowmin_p = jnp.concatenate(
        [rowmin.reshape(B, N), jnp.full((B, padn), 1e30, jnp.float32)], axis=1
    ).reshape(B, 160, 128)
    hits_p = hits.reshape(B, 160, 128)

    red_spec = pl.BlockSpec((1, 160, 128), lambda b: (b, 0, 0))
    out = pl.pallas_call(
        _reduce_body,
        grid=(B,),
        in_specs=[red_spec, red_spec, red_spec],
        out_specs=pl.BlockSpec((1, 128), lambda b: (0, 0)),
        out_shape=jax.ShapeDtypeStruct((1, 128), jnp.float32),
        scratch_shapes=[pltpu.VMEM((1, 128), jnp.float32)],
    )(conf_p, hits_p, rowmin_p)
    return out[0, 0], out[0, 1]


# speedup vs baseline: 5.4012x; 1.3524x over previous
import numpy as np
import jax
import jax.numpy as jnp
from jax import lax
from jax.experimental import pallas as pl
from jax.experimental.pallas import tpu as pltpu
from jax.experimental.pallas import tpu_sc as plsc

_TILE = 1000
_MP = 2048
_NP = 20480
_CLS_OFF = 64.0
_T2 = np.float32((2.5 * np.log(3.0)) ** 2)
_T2_BITS = int(np.float32(_T2).view(np.int32))
_ROW_MASK = 1023
_SC_ROWS = _NP // 32


def _sweep_body(p_ref, g_ref, rowmin_ref, key_ref, wtile_ref, colkey, wtile):
    t = pl.program_id(1)
    nt = pl.num_programs(1)

    @pl.when(t == 0)
    def _reset():
        colkey[...] = jnp.full_like(colkey, jnp.inf)
        wtile[...] = jnp.zeros_like(wtile)

    pv = p_ref[0, 0]
    gv = g_ref[0]

    czv = pv[:, 0:1] * _CLS_OFF
    cxv = pv[:, 1:2]
    cyv = pv[:, 2:3]
    cn2 = cxv * cxv + cyv * cyv
    cnc = cn2 + czv * czv
    onesc = jnp.ones_like(cxv)
    zeroc = jnp.zeros_like(cxv)
    lhsA = jnp.concatenate(
        [czv * -2.0, cxv * -2.0, cyv * -2.0, cnc, onesc, zeroc, zeroc, zeroc],
        axis=1)
    lhsB = jnp.concatenate(
        [zeroc, cxv * -2.0, cyv * -2.0, cn2, zeroc, onesc, zeroc, zeroc],
        axis=1)

    tzv = gv[0:1]
    txv = gv[1:2]
    tyv = gv[2:3]
    tn2 = txv * txv + tyv * tyv
    tnc = tn2 + tzv * tzv
    onest = jnp.ones_like(txv)
    zerot = jnp.zeros_like(txv)
    rhs = jnp.concatenate(
        [tzv, txv, tyv, onest, tnc, tn2, zerot, zerot], axis=0)

    dn = (((1,), (0,)), ((), ()))
    d2 = lax.dot_general(lhsB, rhs, dn,
                         preferred_element_type=jnp.float32)
    rowmin_ref[0, 0] = jnp.min(d2, axis=1, keepdims=True)

    d2c = lax.dot_general(lhsA, rhs, dn,
                          preferred_element_type=jnp.float32)
    lidx = lax.broadcasted_iota(jnp.int32, (_TILE, _MP), 0)
    key = lax.bitcast_convert_type(
        (lax.bitcast_convert_type(d2c, jnp.int32) & jnp.int32(~_ROW_MASK))
        | lidx, jnp.float32)
    kmin = jnp.min(key, axis=0, keepdims=True)
    better = kmin < colkey[...]
    wtile[...] = jnp.where(better, t, wtile[...])
    colkey[...] = jnp.where(better, kmin, colkey[...])

    @pl.when(t == nt - 1)
    def _emit():
        key_ref[0] = lax.bitcast_convert_type(colkey[...], jnp.int32)
        wtile_ref[0] = wtile[...]


def _reduce_body(conf_ref, hits_ref, rowmin_ref, out_ref, acc):
    b = pl.program_id(0)
    nb = pl.num_programs(0)

    @pl.when(b == 0)
    def _init():
        acc[...] = jnp.zeros_like(acc)

    confv = conf_ref[0]
    hitv = hits_ref[0]
    rmv = rowmin_ref[0]
    d = jnp.sqrt(jnp.maximum(rmv, 0.0) + 1e-12)
    sc = 2.0 * jax.nn.sigmoid(-(d / 2.5))
    obj_c = (jnp.sum(jax.nn.softplus(confv))
             - jnp.sum(jnp.where(hitv > 0.0, confv, 0.0)))
    reg_c = jnp.sum(jnp.where(hitv > 0.0, sc, 0.0))

    lane = lax.broadcasted_iota(jnp.int32, (1, 128), 1)
    acc[...] = (acc[...]
                + jnp.where(lane == 0, obj_c, 0.0)
                + jnp.where(lane == 1, reg_c, 0.0))

    @pl.when(b == nb - 1)
    def _emit():
        denom = jnp.asarray(nb * 20000, jnp.float32)
        tot = acc[...]
        out_ref[...] = jnp.where(lane == 0, tot / denom, 1.0 - tot / denom)


def _sc_scatter_body(key_hbm, wtile_hbm, hits_hbm, keybuf, tilebuf, loc):
    wid = lax.axis_index("s") * 2 + lax.axis_index("c")
    lo = wid * _SC_ROWS
    for b in range(4):
        def _zero(i, carry):
            loc[pl.ds(i * 16, 16)] = jnp.zeros((16,), jnp.float32)
            return carry
        lax.fori_loop(0, _SC_ROWS // 16, _zero, 0)
        pltpu.sync_copy(key_hbm.at[b], keybuf)
        pltpu.sync_copy(wtile_hbm.at[b], tilebuf)

        def _scatter(j, carry):
            k = keybuf[pl.ds(j * 16, 16)]
            w = tilebuf[pl.ds(j * 16, 16)]
            n = w * _TILE + (k & _ROW_MASK)
            valid = (k & jnp.int32(~_ROW_MASK)) <= jnp.int32(_T2_BITS)
            inr = valid & (n >= lo) & (n < lo + _SC_ROWS)
            li = jnp.where(inr, n - lo, 0)
            plsc.store_scatter(loc, [li], jnp.ones((16,), jnp.float32),
                               mask=inr)
            return carry
        lax.fori_loop(0, _MP // 16, _scatter, 0)
        pltpu.sync_copy(loc, hits_hbm.at[b, pl.ds(lo, _SC_ROWS)])


def _make_sc_scatter():
    return pl.kernel(
        _sc_scatter_body,
        mesh=plsc.VectorSubcoreMesh(core_axis_name="c", subcore_axis_name="s"),
        out_type=jax.ShapeDtypeStruct((4, _NP), jnp.float32),
        scratch_types=[
            pltpu.VMEM((_MP,), jnp.int32),
            pltpu.VMEM((_MP,), jnp.int32),
            pltpu.VMEM((_SC_ROWS,), jnp.float32),
        ],
        compiler_params=pltpu.CompilerParams(needs_layout_passes=False),
    )


def kernel(pred, gt):
    B, N, _ = pred.shape
    M = gt.shape[1]
    T = N // _TILE

    pv = pred.reshape(B, T, _TILE, 4)
    pad = _MP - M
    gscaled = jnp.stack(
        [gt[:, :, 0] * _CLS_OFF, gt[:, :, 1], gt[:, :, 2]], axis=1)
    gv = jnp.concatenate(
        [gscaled, jnp.full((B, 3, pad), 1e6, jnp.float32)], axis=2)

    rowmin, colkey, wtile = pl.pallas_call(
        _sweep_body,
        grid=(B, T),
        in_specs=[pl.BlockSpec((1, 1, _TILE, 4), lambda b, t: (b, t, 0, 0)),
                  pl.BlockSpec((1, 3, _MP), lambda b, t: (b, 0, 0))],
        out_specs=[pl.BlockSpec((1, 1, _TILE, 1), lambda b, t: (b, t, 0, 0)),
                   pl.BlockSpec((1, 1, _MP), lambda b, t: (b, 0, 0)),
                   pl.BlockSpec((1, 1, _MP), lambda b, t: (b, 0, 0))],
        out_shape=[jax.ShapeDtypeStruct((B, T, _TILE, 1), jnp.float32),
                   jax.ShapeDtypeStruct((B, 1, _MP), jnp.int32),
                   jax.ShapeDtypeStruct((B, 1, _MP), jnp.int32)],
        scratch_shapes=[
            pltpu.VMEM((1, _MP), jnp.float32),
            pltpu.VMEM((1, _MP), jnp.int32),
        ],
    )(pv, gv)

    hits = _make_sc_scatter()(colkey.reshape(B, _MP), wtile.reshape(B, _MP))

    padn = _NP - N
    conf_p = jnp.concatenate(
        [pred[:, :, 3], jnp.full((B, padn), -1e30, jnp.float32)], axis=1
    ).reshape(B, 160, 128)
    rowmin_p = jnp.concatenate(
        [rowmin.reshape(B, N), jnp.full((B, padn), 1e30, jnp.float32)], axis=1
    ).reshape(B, 160, 128)
    hits_p = hits.reshape(B, 160, 128)

    red_spec = pl.BlockSpec((1, 160, 128), lambda b: (b, 0, 0))
    out = pl.pallas_call(
        _reduce_body,
        grid=(B,),
        in_specs=[red_spec, red_spec, red_spec],
        out_specs=pl.BlockSpec((1, 128), lambda b: (0, 0)),
        out_shape=jax.ShapeDtypeStruct((1, 128), jnp.float32),
        scratch_shapes=[pltpu.VMEM((1, 128), jnp.float32)],
    )(conf_p, hits_p, rowmin_p)
    return out[0, 0], out[0, 1]
